# Initial kernel scaffold; baseline (speedup 1.0000x reference)
import jax
import jax.numpy as jnp
from jax.experimental import pallas as pl


def kernel(x, edge_index, U_w, U_b, NN_w, NN_b):
    raise NotImplementedError("write your pallas kernel here")


# speedup vs baseline: 5.6781x; 5.6781x over previous
import functools

import jax
import jax.numpy as jnp
from jax import lax
from jax.experimental import pallas as pl
from jax.experimental.pallas import tpu as pltpu
from jax.experimental.pallas import tpu_sc as plsc

N = 10000
E = 320000
D = 128
DEPTH = 3

NC = 2
NS = 16
NW = NC * NS
CHUNK = 128
NCHUNKS = E // CHUNK
BASE_CHUNKS = NCHUNKS // NW
EXTRA = NCHUNKS - BASE_CHUNKS * NW
ROWS_PER_TILE = 624
TAIL_BASE = ROWS_PER_TILE * 16
ZROWS = 208


def _sc_segment_sum(h, src, dst):
  mesh = plsc.VectorSubcoreMesh(core_axis_name="c", subcore_axis_name="s")

  @functools.partial(
      pl.kernel,
      out_type=jax.ShapeDtypeStruct((NC, N, D), jnp.float32),
      mesh=mesh,
      scratch_types=[
          pltpu.VMEM((CHUNK,), jnp.int32),
          pltpu.VMEM((CHUNK,), jnp.int32),
          pltpu.VMEM((CHUNK, D), jnp.float32),
          pltpu.VMEM((ZROWS, D), jnp.float32),
          pltpu.VMEM_SHARED((N, D), jnp.float32),
          pltpu.SemaphoreType.DMA,
      ],
  )
  def seg_sum(h_hbm, src_hbm, dst_hbm, out_hbm, sidx, didx, rows, zbuf, acc,
              gsem):
    ci = lax.axis_index("c")
    si = lax.axis_index("s")
    wid = ci * NS + si

    zero16 = jnp.zeros((16,), jnp.float32)

    def zrow(i, _):
      for j in range(D // 16):
        zbuf[i, pl.ds(j * 16, 16)] = zero16
      return 0

    lax.fori_loop(0, ZROWS, zrow, 0)
    tile_base = si * ROWS_PER_TILE
    for z in range(ROWS_PER_TILE // ZROWS):
      pltpu.sync_copy(zbuf, acc.at[pl.ds(tile_base + z * ZROWS, ZROWS)])

    @pl.when(si == NS - 1)
    def _():
      pltpu.sync_copy(zbuf.at[pl.ds(0, N - TAIL_BASE)],
                      acc.at[pl.ds(TAIL_BASE, N - TAIL_BASE)])

    plsc.subcore_barrier()

    def do_chunk(c):
      base = c * CHUNK
      pltpu.sync_copy(src_hbm.at[pl.ds(base, CHUNK)], sidx)
      pltpu.sync_copy(dst_hbm.at[pl.ds(base, CHUNK)], didx)
      pltpu.async_copy(h_hbm.at[sidx], rows, gsem).wait()
      pltpu.sync_copy(rows, acc.at[didx], add=True)

    def chunk_body(i, _):
      do_chunk(i * NW + wid)
      return 0

    lax.fori_loop(0, BASE_CHUNKS, chunk_body, 0)

    @pl.when(wid < EXTRA)
    def _():
      do_chunk(BASE_CHUNKS * NW + wid)

    plsc.subcore_barrier()

    pltpu.sync_copy(acc.at[pl.ds(tile_base, ROWS_PER_TILE)],
                    out_hbm.at[ci, pl.ds(tile_base, ROWS_PER_TILE)])

    @pl.when(si == NS - 1)
    def _():
      pltpu.sync_copy(acc.at[pl.ds(TAIL_BASE, N - TAIL_BASE)],
                      out_hbm.at[ci, pl.ds(TAIL_BASE, N - TAIL_BASE)])

  return seg_sum(h, src, dst)


B_R = 2000


def _update_body(h_ref, m_ref, w1_ref, w2_ref, b_ref, o_ref):
  m = m_ref[0] + m_ref[1]
  a = lax.dot_general(h_ref[...], w1_ref[...], (((1,), (1,)), ((), ())),
                      preferred_element_type=jnp.float32,
                      precision=lax.Precision.HIGHEST)
  a = a + lax.dot_general(m, w2_ref[...], (((1,), (1,)), ((), ())),
                          preferred_element_type=jnp.float32,
                          precision=lax.Precision.HIGHEST)
  o_ref[...] = jnp.maximum(a + b_ref[...], 0.0)


def _tc_update(h, m2, w1, w2, b):
  return pl.pallas_call(
      _update_body,
      grid=(N // B_R,),
      in_specs=[
          pl.BlockSpec((B_R, D), lambda i: (i, 0)),
          pl.BlockSpec((NC, B_R, D), lambda i: (0, i, 0)),
          pl.BlockSpec((D, D), lambda i: (0, 0)),
          pl.BlockSpec((D, D), lambda i: (0, 0)),
          pl.BlockSpec((1, D), lambda i: (0, 0)),
      ],
      out_specs=pl.BlockSpec((B_R, D), lambda i: (i, 0)),
      out_shape=jax.ShapeDtypeStruct((N, D), jnp.float32),
  )(h, m2, w1, w2, b)


def _final_body(h_ref, m_ref, w1_ref, w2_ref, b_ref, nnw_ref, nnb_ref, o_ref):
  i = pl.program_id(0)
  m = m_ref[0] + m_ref[1]
  a = lax.dot_general(h_ref[...], w1_ref[...], (((1,), (1,)), ((), ())),
                      preferred_element_type=jnp.float32,
                      precision=lax.Precision.HIGHEST)
  a = a + lax.dot_general(m, w2_ref[...], (((1,), (1,)), ((), ())),
                          preferred_element_type=jnp.float32,
                          precision=lax.Precision.HIGHEST)
  hn = jnp.maximum(a + b_ref[...], 0.0)
  s = jnp.sum(hn, axis=0, keepdims=True)
  p = lax.dot_general(s, nnw_ref[...], (((1,), (1,)), ((), ())),
                      preferred_element_type=jnp.float32,
                      precision=lax.Precision.HIGHEST)

  @pl.when(i == 0)
  def _():
    o_ref[...] = p + nnb_ref[...]

  @pl.when(i > 0)
  def _():
    o_ref[...] = o_ref[...] + p


def _tc_final(h, m2, w1, w2, b, nnw, nnb):
  return pl.pallas_call(
      _final_body,
      grid=(N // B_R,),
      in_specs=[
          pl.BlockSpec((B_R, D), lambda i: (i, 0)),
          pl.BlockSpec((NC, B_R, D), lambda i: (0, i, 0)),
          pl.BlockSpec((D, D), lambda i: (0, 0)),
          pl.BlockSpec((D, D), lambda i: (0, 0)),
          pl.BlockSpec((1, D), lambda i: (0, 0)),
          pl.BlockSpec((1, D), lambda i: (0, 0)),
          pl.BlockSpec((1, 1), lambda i: (0, 0)),
      ],
      out_specs=pl.BlockSpec((1, 1), lambda i: (0, 0)),
      out_shape=jax.ShapeDtypeStruct((1, 1), jnp.float32),
  )(h, m2, w1, w2, b, nnw, nnb)


def kernel(x, edge_index, U_w, U_b, NN_w, NN_b):
  src = edge_index[0]
  dst = edge_index[1]
  w1 = U_w[:, :D]
  w2 = U_w[:, D:]
  b = U_b.reshape(1, D)
  nnb = NN_b.reshape(1, 1)

  h = x
  for step in range(DEPTH):
    m2 = _sc_segment_sum(h, src, dst)
    if step < DEPTH - 1:
      h = _tc_update(h, m2, w1, w2, b)
    else:
      out = _tc_final(h, m2, w1, w2, b, NN_w, nnb)
  return out
